# baseline (speedup 1.0000x reference)
import functools
import math

import jax
import jax.numpy as jnp
from jax import lax
from jax.experimental import pallas as pl
from jax.experimental.pallas import tpu as pltpu
from jax.experimental.pallas import tpu_sc as plsc

NC = 2
NS = 16
L = 16
NW = NC * NS

XB = 4096
S = 200
D = 32

CS = 4
NB = CS * 4
NCHUNK = S // CS
NBUF = 3
FA = 2

SCALE = math.sqrt(32.0)

NE = 1000000
TW = 8192

_mesh = plsc.VectorSubcoreMesh(core_axis_name="c", subcore_axis_name="s")


def _tc_transpose_body(t_ref, o_ref):
    x = t_ref[...]
    eye = (
        lax.broadcasted_iota(jnp.int32, (D, D), 0)
        == lax.broadcasted_iota(jnp.int32, (D, D), 1)
    ).astype(jnp.float32)
    y = lax.dot_general(
        x, eye, (((0,), (0,)), ((), ())),
        preferred_element_type=jnp.float32,
    )
    blk = y.reshape(TW // 4, 4, D)
    o_ref[...] = jnp.concatenate([blk[:, a, :] for a in range(4)], axis=1)


_tc_transpose = pl.pallas_call(
    _tc_transpose_body,
    out_shape=jax.ShapeDtypeStruct((NE // 4, 4 * D), jnp.float32),
    grid=((NE + TW - 1) // TW,),
    in_specs=[pl.BlockSpec((D, TW), lambda i: (0, i))],
    out_specs=pl.BlockSpec((TW // 4, 4 * D), lambda i: (i, 0)),
)


@functools.partial(
    pl.kernel,
    out_type=jax.ShapeDtypeStruct((S * 4 * XB * 8,), jnp.float32),
    mesh=_mesh,
    scratch_types=(
        [pltpu.VMEM((S // 8, 1, 8, 128), jnp.int32)]
        + [pltpu.VMEM((CS * 128, D), jnp.float32) for _ in range(NBUF)]
        + [pltpu.VMEM((NB * 1024,), jnp.float32) for _ in range(NBUF)]
        + [pltpu.SemaphoreType.DMA for _ in range(2 * NBUF)]
    ),
    compiler_params=pltpu.CompilerParams(
        use_tc_tiling_on_sc=False, needs_layout_passes=False
    ),
)
def _gather_scale(idx_hbm, table_hbm, out_hbm, idx_v, *bufs_and_sems):
    rows = bufs_and_sems[:NBUF]
    ostg = bufs_and_sems[NBUF:2 * NBUF]
    gsem = bufs_and_sems[2 * NBUF:3 * NBUF]
    osem = bufs_and_sems[3 * NBUF:]

    wid = lax.axis_index("s") * NC + lax.axis_index("c")

    pltpu.sync_copy(idx_hbm.at[:, pl.ds(wid, 1)], idx_v)

    def fire_gather(i):
        b = i % NBUF
        cps = []
        for si in range(CS):
            s = i * CS + si
            cps.append(
                pltpu.async_copy(
                    table_hbm.at[idx_v.at[s // 8, 0, s % 8]],
                    rows[b].at[pl.ds(si * 128, 128)],
                    gsem[b],
                )
            )
        return cps

    iota16 = lax.iota(jnp.int32, L)
    lane_off = (iota16 >> 3) * 1024 + (iota16 & 7) * 128

    def scale_chunk(rv, ov):
        @plsc.parallel_loop(0, CS * 128, step=1, unroll=4)
        def _(k):
            base = ((k >> 7) << 12) + (k & 127)
            for half in range(2):
                v = rv[k, pl.ds(half * L, L)] * SCALE
                idx = lane_off + (base + half * 2048)
                plsc.store_scatter(ov, [idx], v)

    gdesc = [None] * NCHUNK
    odesc = [None] * NCHUNK
    for i in range(FA):
        gdesc[i] = fire_gather(i)
    for i in range(NCHUNK):
        b = i % NBUF
        f = i + FA
        if f < NCHUNK:
            gdesc[f] = fire_gather(f)
        for cp in gdesc[i]:
            cp.wait()
        if i - NBUF >= 0:
            for cp in odesc[i - NBUF]:
                cp.wait()
        scale_chunk(rows[b], ostg[b])
        odesc[i] = [
            pltpu.async_copy(
                ostg[b].at[pl.ds(j * 1024, 1024)],
                out_hbm.at[pl.ds(((i * NB + j) * 32 + wid) * 1024, 1024)],
                osem[b],
            )
            for j in range(NB)
        ]
    for i in range(NCHUNK - NBUF, NCHUNK):
        for cp in odesc[i]:
            cp.wait()


def kernel(x, table):
    x4 = x.T.reshape(S // 8, 8, XB // 128, 128).transpose(0, 2, 1, 3)
    tlin = _tc_transpose(table.T).reshape(NE, D)
    o1 = _gather_scale(x4, tlin)
    o5 = o1.reshape(S, 4, XB // 128, 8, 128)
    return o5.transpose(2, 4, 0, 1, 3).reshape(XB, S, D)


# speedup vs baseline: 1.0331x; 1.0331x over previous
import functools
import math

import jax
import jax.numpy as jnp
from jax import lax
from jax.experimental import pallas as pl
from jax.experimental.pallas import tpu as pltpu
from jax.experimental.pallas import tpu_sc as plsc

NC = 2
NS = 16
L = 16
NW = NC * NS

XB = 4096
S = 200
D = 32

CS = 4
NB = CS * 4
NCHUNK = S // CS
NBUF = 3
FA = 2

SCALE = math.sqrt(32.0)

NE = 1000000
TW = 8192

_mesh = plsc.VectorSubcoreMesh(core_axis_name="c", subcore_axis_name="s")


def _tc_transpose_body(t_ref, o_ref):
    blk = t_ref[...].T.reshape(TW // 4, 4, D)
    o_ref[...] = jnp.concatenate([blk[:, a, :] for a in range(4)], axis=1)


_tc_transpose = pl.pallas_call(
    _tc_transpose_body,
    out_shape=jax.ShapeDtypeStruct((NE // 4, 4 * D), jnp.float32),
    grid=((NE + TW - 1) // TW,),
    in_specs=[pl.BlockSpec((D, TW), lambda i: (0, i))],
    out_specs=pl.BlockSpec((TW // 4, 4 * D), lambda i: (i, 0)),
)


@functools.partial(
    pl.kernel,
    out_type=jax.ShapeDtypeStruct((S * 4 * XB * 8,), jnp.float32),
    mesh=_mesh,
    scratch_types=(
        [pltpu.VMEM((S // 8, 1, 1024), jnp.int32)]
        + [pltpu.VMEM((CS * 128, D), jnp.float32) for _ in range(NBUF)]
        + [pltpu.VMEM((NB * 1024,), jnp.float32) for _ in range(NBUF)]
        + [pltpu.SemaphoreType.DMA for _ in range(2 * NBUF)]
    ),
    compiler_params=pltpu.CompilerParams(
        use_tc_tiling_on_sc=False, needs_layout_passes=False
    ),
)
def _gather_scale(idx_hbm, table_hbm, out_hbm, idx_v, *bufs_and_sems):
    rows = bufs_and_sems[:NBUF]
    ostg = bufs_and_sems[NBUF:2 * NBUF]
    gsem = bufs_and_sems[2 * NBUF:3 * NBUF]
    osem = bufs_and_sems[3 * NBUF:]

    wid = lax.axis_index("s") * NC + lax.axis_index("c")

    pltpu.sync_copy(idx_hbm.at[:, pl.ds(wid, 1)], idx_v)

    def fire_gather(i):
        b = i % NBUF
        s0 = i * CS
        return [
            pltpu.async_copy(
                table_hbm.at[
                    idx_v.at[s0 // 8, 0, pl.ds((s0 % 8) * 128, CS * 128)]
                ],
                rows[b],
                gsem[b],
            )
        ]

    iota16 = lax.iota(jnp.int32, L)
    lane_off = (iota16 >> 3) * 1024 + (iota16 & 7) * 128

    def scale_chunk(rv, ov):
        @plsc.parallel_loop(0, CS * 128, step=1, unroll=4)
        def _(k):
            base = ((k >> 7) << 12) + (k & 127)
            for half in range(2):
                v = rv[k, pl.ds(half * L, L)] * SCALE
                idx = lane_off + (base + half * 2048)
                plsc.store_scatter(ov, [idx], v)

    gdesc = [None] * NCHUNK
    odesc = [None] * NCHUNK
    for i in range(FA):
        gdesc[i] = fire_gather(i)
    for i in range(NCHUNK):
        b = i % NBUF
        f = i + FA
        if f < NCHUNK:
            gdesc[f] = fire_gather(f)
        for cp in gdesc[i]:
            cp.wait()
        if i - NBUF >= 0:
            for cp in odesc[i - NBUF]:
                cp.wait()
        scale_chunk(rows[b], ostg[b])
        odesc[i] = [
            pltpu.async_copy(
                ostg[b].at[pl.ds(j * 1024, 1024)],
                out_hbm.at[pl.ds(((i * NB + j) * 32 + wid) * 1024, 1024)],
                osem[b],
            )
            for j in range(NB)
        ]
    for i in range(NCHUNK - NBUF, NCHUNK):
        for cp in odesc[i]:
            cp.wait()


def kernel(x, table):
    x4 = (
        x.T.reshape(S // 8, 8, XB // 128, 128)
        .transpose(0, 2, 1, 3)
        .reshape(S // 8, XB // 128, 1024)
    )
    tlin = _tc_transpose(table.T).reshape(NE, D)
    o1 = _gather_scale(x4, tlin)
    o5 = o1.reshape(S, 4, XB // 128, 8, 128)
    return o5.transpose(2, 4, 0, 1, 3).reshape(XB, S, D)
